# baseline (speedup 1.0000x reference)
import functools

import jax
import jax.numpy as jnp
from jax import lax
from jax.experimental import pallas as pl
from jax.experimental.pallas import tpu as pltpu
from jax.experimental.pallas import tpu_sc as plsc

_INFO = plsc.get_sparse_core_info()
_NC = _INFO.num_cores
_NS = _INFO.num_subcores
_NW = _NC * _NS

_CHUNK = 1024


def _make_gather(n_rows: int, dim: int):
    assert n_rows % (_NW * _CHUNK) == 0
    b_per_w = n_rows // _NW
    n_chunks = b_per_w // _CHUNK
    mesh = plsc.VectorSubcoreMesh(core_axis_name="c", subcore_axis_name="s")

    @functools.partial(
        pl.kernel,
        mesh=mesh,
        out_type=jax.ShapeDtypeStruct((n_rows, dim), jnp.float32),
        scratch_types=[
            pltpu.VMEM((_CHUNK,), jnp.int32),
            pltpu.VMEM((_CHUNK, dim), jnp.float32),
            pltpu.SemaphoreType.DMA,
        ],
        compiler_params=pltpu.CompilerParams(use_tc_tiling_on_sc=False),
    )
    def gather_kernel(idx_hbm, table_hbm, out_hbm, idx_v, rows_v, sem):
        wid = lax.axis_index("s") * _NC + lax.axis_index("c")
        base = wid * b_per_w

        def body(i, carry):
            off = base + i * _CHUNK
            pltpu.sync_copy(idx_hbm.at[pl.ds(off, _CHUNK)], idx_v)
            pltpu.async_copy(table_hbm.at[idx_v], rows_v, sem).wait()
            pltpu.sync_copy(rows_v, out_hbm.at[pl.ds(off, _CHUNK)])
            return carry

        lax.fori_loop(0, n_chunks, body, 0)

    return gather_kernel


def kernel(idxes, table):
    batch, seq = idxes.shape
    dim = table.shape[1]
    flat = idxes.reshape(batch * seq)
    out = _make_gather(batch * seq, dim)(flat, table)
    return out.reshape(batch, seq, dim)


# speedup vs baseline: 1.0113x; 1.0113x over previous
import functools

import jax
import jax.numpy as jnp
from jax import lax
from jax.experimental import pallas as pl
from jax.experimental.pallas import tpu as pltpu
from jax.experimental.pallas import tpu_sc as plsc

_INFO = plsc.get_sparse_core_info()
_NC = _INFO.num_cores
_NS = _INFO.num_subcores
_NW = _NC * _NS

_CHUNK = 512
_NBUF = 2


def _make_gather(n_rows: int, dim: int):
    assert n_rows % (_NW * _CHUNK * _NBUF) == 0
    b_per_w = n_rows // _NW
    n_groups = b_per_w // (_CHUNK * _NBUF)
    mesh = plsc.VectorSubcoreMesh(core_axis_name="c", subcore_axis_name="s")

    @functools.partial(
        pl.kernel,
        mesh=mesh,
        out_type=jax.ShapeDtypeStruct((n_rows, dim), jnp.float32),
        scratch_types=[
            pltpu.VMEM((b_per_w,), jnp.int32),
            [pltpu.VMEM((_CHUNK, dim), jnp.float32) for _ in range(_NBUF)],
            [pltpu.SemaphoreType.DMA for _ in range(_NBUF)],
            [pltpu.SemaphoreType.DMA for _ in range(_NBUF)],
        ],
        compiler_params=pltpu.CompilerParams(use_tc_tiling_on_sc=False),
    )
    def gather_kernel(idx_hbm, table_hbm, out_hbm, idx_v, rows, gsems, wsems):
        wid = lax.axis_index("s") * _NC + lax.axis_index("c")
        base = wid * b_per_w
        pltpu.sync_copy(idx_hbm.at[pl.ds(base, b_per_w)], idx_v)

        def group(g, carry):
            gathers = []
            for b in range(_NBUF):
                i = g * _NBUF + b
                off = i * _CHUNK

                @pl.when(g > 0)
                def _(b=b, off=off):
                    pltpu.make_async_copy(
                        rows[b], out_hbm.at[pl.ds(base + off, _CHUNK)], wsems[b]
                    ).wait()

                gathers.append(
                    pltpu.async_copy(
                        table_hbm.at[idx_v.at[pl.ds(off, _CHUNK)]],
                        rows[b],
                        gsems[b],
                    )
                )
            for b in range(_NBUF):
                i = g * _NBUF + b
                off = i * _CHUNK
                gathers[b].wait()
                pltpu.async_copy(
                    rows[b], out_hbm.at[pl.ds(base + off, _CHUNK)], wsems[b]
                )
            return carry

        lax.fori_loop(0, n_groups, group, 0)

        for b in range(_NBUF):
            off = ((n_groups - 1) * _NBUF + b) * _CHUNK
            pltpu.make_async_copy(
                rows[b], out_hbm.at[pl.ds(base + off, _CHUNK)], wsems[b]
            ).wait()

    return gather_kernel


def kernel(idxes, table):
    batch, seq = idxes.shape
    dim = table.shape[1]
    flat = idxes.reshape(batch * seq)
    out = _make_gather(batch * seq, dim)(flat, table)
    return out.reshape(batch, seq, dim)


# speedup vs baseline: 1.0165x; 1.0052x over previous
import functools

import jax
import jax.numpy as jnp
from jax import lax
from jax.experimental import pallas as pl
from jax.experimental.pallas import tpu as pltpu
from jax.experimental.pallas import tpu_sc as plsc

_INFO = plsc.get_sparse_core_info()
_NC = _INFO.num_cores
_NS = _INFO.num_subcores
_NW = _NC * _NS

_CB = 4
_NBUF = 2


def _make_gather(batch: int, seq: int, dim: int):
    assert batch % (_NW * _CB * _NBUF) == 0
    rows_per_w = batch // _NW
    n_groups = rows_per_w // (_CB * _NBUF)
    chunk = _CB * seq
    mesh = plsc.VectorSubcoreMesh(core_axis_name="c", subcore_axis_name="s")

    @functools.partial(
        pl.kernel,
        mesh=mesh,
        out_type=jax.ShapeDtypeStruct((batch * seq, dim), jnp.float32),
        scratch_types=[
            [pltpu.VMEM((_CB, seq), jnp.int32) for _ in range(_NBUF)],
            [pltpu.VMEM((chunk, dim), jnp.float32) for _ in range(_NBUF)],
            [pltpu.SemaphoreType.DMA for _ in range(_NBUF)],
            [pltpu.SemaphoreType.DMA for _ in range(_NBUF)],
        ],
        compiler_params=pltpu.CompilerParams(use_tc_tiling_on_sc=False),
    )
    def gather_kernel(idx_hbm, table_hbm, out_hbm, idx_v, rows, gsems, wsems):
        wid = lax.axis_index("s") * _NC + lax.axis_index("c")
        base_row = wid * rows_per_w

        def group(g, carry):
            gathers = []
            for b in range(_NBUF):
                r0 = base_row + (g * _NBUF + b) * _CB

                @pl.when(g > 0)
                def _(b=b, r0=r0):
                    pltpu.make_async_copy(
                        rows[b], out_hbm.at[pl.ds(r0 * seq, chunk)], wsems[b]
                    ).wait()

                pltpu.sync_copy(idx_hbm.at[pl.ds(r0, _CB)], idx_v[b])
                for j in range(_CB):
                    gathers.append(
                        pltpu.async_copy(
                            table_hbm.at[idx_v[b].at[j]],
                            rows[b].at[pl.ds(j * seq, seq)],
                            gsems[b],
                        )
                    )
            for b in range(_NBUF):
                r0 = base_row + (g * _NBUF + b) * _CB
                for j in range(_CB):
                    gathers[b * _CB + j].wait()
                pltpu.async_copy(
                    rows[b], out_hbm.at[pl.ds(r0 * seq, chunk)], wsems[b]
                )
            return carry

        lax.fori_loop(0, n_groups, group, 0)

        for b in range(_NBUF):
            r0 = base_row + ((n_groups - 1) * _NBUF + b) * _CB
            pltpu.make_async_copy(
                rows[b], out_hbm.at[pl.ds(r0 * seq, chunk)], wsems[b]
            ).wait()

    return gather_kernel


def kernel(idxes, table):
    batch, seq = idxes.shape
    dim = table.shape[1]
    out = _make_gather(batch, seq, dim)(idxes, table)
    return out.reshape(batch, seq, dim)


# speedup vs baseline: 1.4495x; 1.4259x over previous
import functools

import jax
import jax.numpy as jnp
from jax import lax
from jax.experimental import pallas as pl
from jax.experimental.pallas import tpu as pltpu
from jax.experimental.pallas import tpu_sc as plsc

_INFO = plsc.get_sparse_core_info()
_NC = _INFO.num_cores
_NS = _INFO.num_subcores
_NW = _NC * _NS

_CB = 4
_NBUF = 2


def _make_gather(batch: int, seq: int, dim: int):
    assert batch % (_NW * _CB * _NBUF) == 0
    rows_per_w = batch // _NW
    n_groups = rows_per_w // (_CB * _NBUF)
    chunk = _CB * seq
    mesh = plsc.VectorSubcoreMesh(core_axis_name="c", subcore_axis_name="s")

    @functools.partial(
        pl.kernel,
        mesh=mesh,
        out_type=jax.ShapeDtypeStruct((batch * seq, 2 * dim), jnp.float32),
        scratch_types=[
            [pltpu.VMEM((_CB, seq), jnp.int32) for _ in range(_NBUF)],
            [pltpu.VMEM((chunk, dim), jnp.float32) for _ in range(_NBUF)],
            [pltpu.SemaphoreType.DMA for _ in range(_NBUF)],
            [pltpu.SemaphoreType.DMA for _ in range(_NBUF)],
        ],
        compiler_params=pltpu.CompilerParams(use_tc_tiling_on_sc=False),
    )
    def gather_kernel(idx_hbm, table_hbm, out_hbm, idx_v, rows, gsems, wsems):
        wid = lax.axis_index("s") * _NC + lax.axis_index("c")
        base_row = wid * rows_per_w

        def group(g, carry):
            gathers = []
            for b in range(_NBUF):
                r0 = base_row + (g * _NBUF + b) * _CB

                @pl.when(g > 0)
                def _(b=b, r0=r0):
                    pltpu.make_async_copy(
                        rows[b],
                        out_hbm.at[pl.ds(r0 * seq, chunk), pl.ds(0, dim)],
                        wsems[b],
                    ).wait()

                pltpu.sync_copy(idx_hbm.at[pl.ds(r0, _CB)], idx_v[b])
                for j in range(_CB):
                    gathers.append(
                        pltpu.async_copy(
                            table_hbm.at[idx_v[b].at[j]],
                            rows[b].at[pl.ds(j * seq, seq)],
                            gsems[b],
                        )
                    )
            for b in range(_NBUF):
                r0 = base_row + (g * _NBUF + b) * _CB
                for j in range(_CB):
                    gathers[b * _CB + j].wait()
                pltpu.async_copy(
                    rows[b],
                    out_hbm.at[pl.ds(r0 * seq, chunk), pl.ds(0, dim)],
                    wsems[b],
                )
            return carry

        lax.fori_loop(0, n_groups, group, 0)

        for b in range(_NBUF):
            r0 = base_row + ((n_groups - 1) * _NBUF + b) * _CB
            pltpu.make_async_copy(
                rows[b],
                out_hbm.at[pl.ds(r0 * seq, chunk), pl.ds(0, dim)],
                wsems[b],
            ).wait()

    return gather_kernel


def kernel(idxes, table):
    batch, seq = idxes.shape
    vocab, dim = table.shape
    table2 = jnp.pad(table, ((0, 0), (0, 64))).reshape(2 * vocab, dim)
    out = _make_gather(batch, seq, dim)(idxes * 2, table2)
    return out.reshape(batch, seq, 2 * dim)[:, :, :dim]
